# baseline (speedup 1.0000x reference)
import functools

import jax
import jax.numpy as jnp
from jax import lax
from jax.experimental import pallas as pl
from jax.experimental.pallas import tpu as pltpu
from jax.experimental.pallas import tpu_sc as plsc

_NC = 2
_NS = 16
_NW = _NC * _NS

_D = 32
_O = 20
_K = 10
_SLOTS = 224
_G = _SLOTS // 16
_HALF = _SLOTS // 2
_NBUF = 4


def _sc_dots(center_vectors, outside_vectors, idx_flat, center_idx):
    B = center_idx.shape[0]
    epw = B // _NW

    mesh = plsc.VectorSubcoreMesh(core_axis_name="c", subcore_axis_name="s")

    @functools.partial(
        pl.kernel,
        out_type=jax.ShapeDtypeStruct((B * _SLOTS,), jnp.float32),
        mesh=mesh,
        compiler_params=pltpu.CompilerParams(
            needs_layout_passes=False, use_tc_tiling_on_sc=False),
        scratch_types=[
            pltpu.VMEM((epw * _SLOTS,), jnp.int32),
            pltpu.VMEM((epw,), jnp.int32),
            pltpu.VMEM((epw, _D), jnp.float32),
            pltpu.VMEM((_NBUF, _SLOTS, _D), jnp.float32),
            pltpu.VMEM((epw * _SLOTS,), jnp.float32),
            pltpu.SemaphoreType.DMA,
        ] + [pltpu.SemaphoreType.DMA] * _NBUF,
    )
    def k(cvec_hbm, ovec_hbm, idx_hbm, cidx_hbm, out_hbm,
          idx_v, cidx_v, z_v, rows_v, out_v, semz, *sems):
        wid = lax.axis_index("s") * _NC + lax.axis_index("c")
        ebase = wid * epw

        pltpu.sync_copy(idx_hbm.at[pl.ds(ebase * _SLOTS, epw * _SLOTS)], idx_v)
        pltpu.sync_copy(cidx_hbm.at[pl.ds(ebase, epw)], cidx_v)
        pltpu.async_copy(cvec_hbm.at[cidx_v], z_v, semz).wait()

        def start(e, buf, sem):
            for h in range(2):
                pltpu.async_copy(
                    ovec_hbm.at[idx_v.at[pl.ds(e * _SLOTS + h * _HALF, _HALF)]],
                    rows_v.at[buf, pl.ds(h * _HALF, _HALF)],
                    sem,
                )

        def drain(buf, sem):
            for h in range(2):
                pltpu.make_async_copy(
                    ovec_hbm.at[pl.ds(0, _HALF)],
                    rows_v.at[buf, pl.ds(h * _HALF, _HALF)],
                    sem,
                ).wait()

        iota16 = lax.iota(jnp.int32, 16)
        row_ids = [iota16 + g * 16 for g in range(_G)]

        def compute(e, buf):
            rows = rows_v.at[buf]
            zhalves = [z_v[e, pl.ds(0, 16)], z_v[e, pl.ds(16, 16)]]
            accs = [jnp.zeros((16,), jnp.float32)] * _G
            for d in range(_D):
                zd = zhalves[d // 16][d % 16]
                dvec = jnp.full((16,), d, jnp.int32)
                accs = [
                    accs[g] + plsc.load_gather(rows, [row_ids[g], dvec]) * zd
                    for g in range(_G)
                ]
            for g in range(_G):
                out_v[pl.ds(e * _SLOTS + g * 16, 16)] = accs[g]

        for b in range(_NBUF):
            start(b, b, sems[b])

        def body(i, carry):
            for sub in range(_NBUF):
                e = _NBUF * i + sub
                drain(sub, sems[sub])
                compute(e, sub)

                @pl.when(i < epw // _NBUF - 1)
                def _():
                    start(e + _NBUF, sub, sems[sub])
            return carry

        lax.fori_loop(0, epw // _NBUF, body, 0)

        pltpu.sync_copy(out_v, out_hbm.at[pl.ds(ebase * _SLOTS, epw * _SLOTS)])

    return k(center_vectors, outside_vectors, idx_flat, center_idx)


def _tc_loss(dots, idx_all, own_all):
    B = dots.shape[0]
    tb = 512

    def body(dots_ref, idx_ref, own_ref, out_ref):
        d = dots_ref[...]
        valid = (idx_ref[...] != 0) & (own_ref[...] != 0)
        col = lax.broadcasted_iota(jnp.int32, d.shape, 1)
        t = jnp.where(col < _O, d, -d)
        ls = jnp.minimum(t, 0.0) - jnp.log1p(jnp.exp(-jnp.abs(t)))
        out_ref[...] = -jnp.sum(jnp.where(valid, ls, 0.0), axis=1)

    return pl.pallas_call(
        body,
        out_shape=jax.ShapeDtypeStruct((B,), jnp.float32),
        grid=(B // tb,),
        in_specs=[
            pl.BlockSpec((tb, _SLOTS), lambda i: (i, 0)),
            pl.BlockSpec((tb, _SLOTS), lambda i: (i, 0)),
            pl.BlockSpec((tb, _SLOTS), lambda i: (i, 0)),
        ],
        out_specs=pl.BlockSpec((tb,), lambda i: (i,)),
    )(dots, idx_all, own_all)


def kernel(center_vectors, outside_vectors, center_word_index,
           outside_word_indices, negative_samples):
    B, O = outside_word_indices.shape
    K = negative_samples.shape[-1]

    neg2 = negative_samples.reshape(B, O * K)
    pad = jnp.zeros((B, _SLOTS - O - O * K), jnp.int32)
    idx_all = jnp.concatenate([outside_word_indices, neg2, pad], axis=1)
    own_all = jnp.concatenate(
        [outside_word_indices,
         jnp.repeat(outside_word_indices, K, axis=1),
         pad],
        axis=1,
    )

    dots = _sc_dots(center_vectors, outside_vectors,
                    idx_all.reshape(-1), center_word_index)
    return _tc_loss(dots.reshape(B, _SLOTS), idx_all, own_all)


# speedup vs baseline: 1.0033x; 1.0033x over previous
import functools

import jax
import jax.numpy as jnp
from jax import lax
from jax.experimental import pallas as pl
from jax.experimental.pallas import tpu as pltpu
from jax.experimental.pallas import tpu_sc as plsc

_NC = 2
_NS = 16
_NW = _NC * _NS

_D = 32
_O = 20
_K = 10
_SLOTS = 224
_G = _SLOTS // 16
_HALF = _SLOTS // 2
_NBUF = 4


_TCH = 128


def _sc_relayout(ovT, cvT, ov_tail, cv_tail):
    Dd, N = ovT.shape
    nfull = N // _TCH
    ntail = N - nfull * _TCH
    per = nfull // _NW
    extra = nfull - per * _NW

    mesh = plsc.VectorSubcoreMesh(core_axis_name="c", subcore_axis_name="s")

    @functools.partial(
        pl.kernel,
        out_type=(jax.ShapeDtypeStruct((N * Dd,), jnp.float32),
                  jax.ShapeDtypeStruct((N * Dd,), jnp.float32)),
        mesh=mesh,
        compiler_params=pltpu.CompilerParams(
            needs_layout_passes=False, use_tc_tiling_on_sc=True),
        scratch_types=[
            pltpu.VMEM((4, Dd, _TCH), jnp.float32),
            pltpu.VMEM((4, _TCH * Dd), jnp.float32),
            pltpu.VMEM((ntail * Dd,), jnp.float32),
        ] + [pltpu.SemaphoreType.DMA] * 8,
    )
    def k(ovT_hbm, cvT_hbm, ovt_hbm, cvt_hbm, ov_out, cv_out,
          in_v, out_v, tail_v, *sems):
        wid = lax.axis_index("s") * _NC + lax.axis_index("c")
        iota16 = lax.iota(jnp.int32, 16)

        srcs = (ovT_hbm, cvT_hbm)
        dsts = (ov_out, cv_out)

        def chunk_of(j):
            return j * _NW + wid

        def start_in(t, j, buf):
            c = chunk_of(j)
            pltpu.async_copy(
                srcs[t].at[:, pl.ds(c * _TCH, _TCH)],
                in_v.at[buf], sems[buf])

        def transpose_chunk(t, j, buf):
            c = chunk_of(j)
            @pl.when(j >= 2)
            def _():
                pltpu.make_async_copy(
                    dsts[t].at[pl.ds(0, _TCH * Dd)],
                    out_v.at[buf], sems[4 + buf]).wait()
            pltpu.make_async_copy(
                srcs[t].at[:, pl.ds(0, _TCH)], in_v.at[buf], sems[buf]).wait()
            src = in_v.at[buf]

            def grp(g, carry):
                for ti in range(8):
                    tok = g * 8 + ti
                    for h in range(2):
                        v = plsc.load_gather(
                            src, [iota16 + 16 * h, jnp.full((16,), tok, jnp.int32)])
                        out_v[buf, pl.ds(tok * Dd + 16 * h, 16)] = v
                return carry

            lax.fori_loop(0, _TCH // 8, grp, 0)
            pltpu.async_copy(
                out_v.at[buf],
                dsts[t].at[pl.ds(c * _TCH * Dd, _TCH * Dd)], sems[4 + buf])

        my_full = per

        start_in(0, 0, 0)
        start_in(1, 0, 2)
        start_in(0, 1, 1)
        start_in(1, 1, 3)

        def body(i, carry):
            for sub in range(2):
                j = 2 * i + sub
                transpose_chunk(0, j, sub)
                transpose_chunk(1, j, 2 + sub)

                @pl.when(j + 2 < my_full)
                def _():
                    start_in(0, j + 2, sub)
                    start_in(1, j + 2, 2 + sub)
            return carry

        lax.fori_loop(0, my_full // 2, body, 0)

        for buf in range(4):
            pltpu.make_async_copy(
                dsts[0].at[pl.ds(0, _TCH * Dd)], out_v.at[buf],
                sems[4 + buf]).wait()

        @pl.when(wid < extra)
        def _():
            for t in range(2):
                buf = 2 * t
                c0 = nfull - extra
                pltpu.async_copy(
                    srcs[t].at[:, pl.ds((c0 + wid) * _TCH, _TCH)],
                    in_v.at[buf], sems[buf])
                pltpu.make_async_copy(
                    srcs[t].at[:, pl.ds(0, _TCH)], in_v.at[buf],
                    sems[buf]).wait()
                src = in_v.at[buf]

                def grp2(g, carry):
                    for ti in range(8):
                        tok = g * 8 + ti
                        for h in range(2):
                            v = plsc.load_gather(
                                src,
                                [iota16 + 16 * h, jnp.full((16,), tok, jnp.int32)])
                            out_v[buf, pl.ds(tok * Dd + 16 * h, 16)] = v
                    return carry

                lax.fori_loop(0, _TCH // 8, grp2, 0)
                pltpu.sync_copy(
                    out_v.at[buf],
                    dsts[t].at[pl.ds((c0 + wid) * _TCH * Dd, _TCH * Dd)])

        tails = (ovt_hbm, cvt_hbm)
        for t in range(2):
            @pl.when(wid == 4 + t)
            def _():
                pltpu.sync_copy(tails[t], tail_v)

                def tgrp(g, carry):
                    for ti in range(4):
                        tok = g * 4 + ti
                        for h in range(2):
                            v = plsc.load_gather(
                                tail_v,
                                [iota16 * ntail + (16 * h * ntail + tok)])
                            out_v[0, pl.ds(tok * Dd + 16 * h, 16)] = v
                    return carry

                lax.fori_loop(0, ntail // 4, tgrp, 0)
                pltpu.sync_copy(
                    out_v.at[0, pl.ds(0, ntail * Dd)],
                    dsts[t].at[pl.ds(nfull * _TCH * Dd, ntail * Dd)])

    return k(ovT, cvT, ov_tail, cv_tail)


def _sc_dots(center_vectors, outside_vectors, idx_flat, center_idx):
    B = center_idx.shape[0]
    epw = B // _NW

    mesh = plsc.VectorSubcoreMesh(core_axis_name="c", subcore_axis_name="s")

    @functools.partial(
        pl.kernel,
        out_type=jax.ShapeDtypeStruct((B * _SLOTS,), jnp.float32),
        mesh=mesh,
        compiler_params=pltpu.CompilerParams(
            needs_layout_passes=False, use_tc_tiling_on_sc=False),
        scratch_types=[
            pltpu.VMEM((epw * _SLOTS,), jnp.int32),
            pltpu.VMEM((epw,), jnp.int32),
            pltpu.VMEM((epw, _D), jnp.float32),
            pltpu.VMEM((_NBUF, _SLOTS, _D), jnp.float32),
            pltpu.VMEM((epw * _SLOTS,), jnp.float32),
            pltpu.SemaphoreType.DMA,
        ] + [pltpu.SemaphoreType.DMA] * _NBUF,
    )
    def k(cvec_hbm, ovec_hbm, idx_hbm, cidx_hbm, out_hbm,
          idx_v, cidx_v, z_v, rows_v, out_v, semz, *sems):
        wid = lax.axis_index("s") * _NC + lax.axis_index("c")
        ebase = wid * epw

        pltpu.sync_copy(idx_hbm.at[pl.ds(ebase * _SLOTS, epw * _SLOTS)], idx_v)
        pltpu.sync_copy(cidx_hbm.at[pl.ds(ebase, epw)], cidx_v)
        pltpu.async_copy(cvec_hbm.at[cidx_v], z_v, semz).wait()

        def start(e, buf, sem):
            for h in range(2):
                pltpu.async_copy(
                    ovec_hbm.at[idx_v.at[pl.ds(e * _SLOTS + h * _HALF, _HALF)]],
                    rows_v.at[buf, pl.ds(h * _HALF, _HALF)],
                    sem,
                )

        def drain(buf, sem):
            for h in range(2):
                pltpu.make_async_copy(
                    ovec_hbm.at[pl.ds(0, _HALF)],
                    rows_v.at[buf, pl.ds(h * _HALF, _HALF)],
                    sem,
                ).wait()

        iota16 = lax.iota(jnp.int32, 16)
        row_ids = [iota16 + g * 16 for g in range(_G)]

        def compute(e, buf):
            rows = rows_v.at[buf]
            zhalves = [z_v[e, pl.ds(0, 16)], z_v[e, pl.ds(16, 16)]]
            accs = [jnp.zeros((16,), jnp.float32)] * _G
            for d in range(_D):
                zd = zhalves[d // 16][d % 16]
                dvec = jnp.full((16,), d, jnp.int32)
                accs = [
                    accs[g] + plsc.load_gather(rows, [row_ids[g], dvec]) * zd
                    for g in range(_G)
                ]
            for g in range(_G):
                out_v[pl.ds(e * _SLOTS + g * 16, 16)] = accs[g]

        for b in range(_NBUF):
            start(b, b, sems[b])

        def body(i, carry):
            for sub in range(_NBUF):
                e = _NBUF * i + sub
                drain(sub, sems[sub])
                compute(e, sub)

                @pl.when(i < epw // _NBUF - 1)
                def _():
                    start(e + _NBUF, sub, sems[sub])
            return carry

        lax.fori_loop(0, epw // _NBUF, body, 0)

        pltpu.sync_copy(out_v, out_hbm.at[pl.ds(ebase * _SLOTS, epw * _SLOTS)])

    return k(center_vectors, outside_vectors, idx_flat, center_idx)


def _tc_loss(dots, idx_all):
    B = dots.shape[0]
    tb = 512

    def body(dots_ref, idx_ref, out_ref):
        d = dots_ref[...]
        valid = idx_ref[...] != 0
        col = lax.broadcasted_iota(jnp.int32, d.shape, 1)
        t = jnp.where(col < _O, d, -d)
        ls = jnp.minimum(t, 0.0) - jnp.log1p(jnp.exp(-jnp.abs(t)))
        out_ref[...] = -jnp.sum(jnp.where(valid, ls, 0.0), axis=1)

    return pl.pallas_call(
        body,
        out_shape=jax.ShapeDtypeStruct((B,), jnp.float32),
        grid=(B // tb,),
        in_specs=[
            pl.BlockSpec((tb, _SLOTS), lambda i: (i, 0)),
            pl.BlockSpec((tb, _SLOTS), lambda i: (i, 0)),
        ],
        out_specs=pl.BlockSpec((tb,), lambda i: (i,)),
    )(dots, idx_all)


def kernel(center_vectors, outside_vectors, center_word_index,
           outside_word_indices, negative_samples):
    B, O = outside_word_indices.shape
    K = negative_samples.shape[-1]

    neg2 = jnp.where(outside_word_indices[:, :, None] == 0, 0,
                     negative_samples).reshape(B, O * K)
    pad = jnp.zeros((B, _SLOTS - O - O * K), jnp.int32)
    idx_all = jnp.concatenate([outside_word_indices, neg2, pad], axis=1)

    N, Dd = outside_vectors.shape
    nfull = (N // _TCH) * _TCH
    ovT = outside_vectors.T
    cvT = center_vectors.T
    ov_flat, cv_flat = _sc_relayout(
        ovT, cvT,
        ovT[:, nfull:].reshape(-1), cvT[:, nfull:].reshape(-1))
    ov_lin = ov_flat.reshape(N, Dd)
    cv_lin = cv_flat.reshape(N, Dd)

    dots = _sc_dots(cv_lin, ov_lin, idx_all.reshape(-1), center_word_index)
    return _tc_loss(dots.reshape(B, _SLOTS), idx_all)
